# initial kernel scaffold (unmeasured)
import jax
import jax.numpy as jnp
from jax import lax
from jax.experimental import pallas as pl
from jax.experimental.pallas import tpu as pltpu


def kernel(
    x,
):
    def body(*refs):
        pass

    out_shape = jax.ShapeDtypeStruct(..., jnp.float32)
    return pl.pallas_call(body, out_shape=out_shape)(...)


# baseline (device time: 28177 ns/iter reference)
import jax
import jax.numpy as jnp
from jax import lax
from jax.experimental import pallas as pl
from jax.experimental.pallas import tpu as pltpu


def kernel(x):
    m, n = x.shape[-2], x.shape[-1]
    x2 = x.reshape(m, n)

    def body(x_ref, out_ref, send_buf, recv_buf, send_sems, recv_sems):
        my_x = lax.axis_index("x")
        my_y = lax.axis_index("y")
        my_z = lax.axis_index("z")

        peers = [
            (my_x, my_y, 1 - my_z),
            (my_x, 1 - my_y, my_z),
            (1 - my_x, my_y, my_z),
        ]

        barrier_sem = pltpu.get_barrier_semaphore()
        for peer in peers:
            pl.semaphore_signal(
                barrier_sem, inc=1,
                device_id=peer, device_id_type=pl.DeviceIdType.MESH,
            )
        pl.semaphore_wait(barrier_sem, len(peers))

        out_ref[...] = x_ref[...]

        for ph, peer in enumerate(peers):
            send_buf[...] = out_ref[...].astype(jnp.bfloat16)
            rdma = pltpu.make_async_remote_copy(
                src_ref=send_buf,
                dst_ref=recv_buf.at[ph],
                send_sem=send_sems.at[ph],
                recv_sem=recv_sems.at[ph],
                device_id=peer,
                device_id_type=pl.DeviceIdType.MESH,
            )
            rdma.start()
            rdma.wait()
            out_ref[...] += recv_buf[ph].astype(jnp.float32)

    return pl.pallas_call(
        body,
        out_shape=jax.ShapeDtypeStruct((m, n), jnp.float32),
        in_specs=[pl.BlockSpec(memory_space=pltpu.VMEM)],
        out_specs=pl.BlockSpec(memory_space=pltpu.VMEM),
        scratch_shapes=[
            pltpu.VMEM((m, n), jnp.bfloat16),
            pltpu.VMEM((3, m, n), jnp.bfloat16),
            pltpu.SemaphoreType.DMA((3,)),
            pltpu.SemaphoreType.DMA((3,)),
        ],
        compiler_params=pltpu.CompilerParams(collective_id=0),
    )(x2)


# device time: 16968 ns/iter; 1.6606x vs baseline; 1.6606x over previous
import jax
import jax.numpy as jnp
from jax import lax
from jax.experimental import pallas as pl
from jax.experimental.pallas import tpu as pltpu

CHUNKS = ((0, 176), (176, 176), (352, 160))
N_STEPS = 3


def kernel(x):
    m, n = x.shape[-2], x.shape[-1]
    x2 = x.reshape(m, n)

    def body(x_ref, out_ref, acc, recv, send_sems, recv_sems):
        me = (lax.axis_index("x"), lax.axis_index("y"), lax.axis_index("z"))

        def peer_along(a):
            p = list(me)
            p[a] = 1 - p[a]
            return tuple(p)

        barrier_sem = pltpu.get_barrier_semaphore()
        for a in range(3):
            pl.semaphore_signal(
                barrier_sem, inc=1,
                device_id=peer_along(a), device_id_type=pl.DeviceIdType.MESH,
            )
        pl.semaphore_wait(barrier_sem, 3)

        acc[...] = x_ref[...].astype(jnp.bfloat16)

        def make_rdma(s, c):
            off, ln = CHUNKS[c]
            return pltpu.make_async_remote_copy(
                src_ref=acc.at[pl.ds(off, ln)],
                dst_ref=recv.at[s, pl.ds(off, ln)],
                send_sem=send_sems.at[s, c],
                recv_sem=recv_sems.at[s, c],
                device_id=peer_along((c + s) % 3),
                device_id_type=pl.DeviceIdType.MESH,
            )

        rdmas = {}
        for c in range(3):
            rdmas[(0, c)] = make_rdma(0, c)
            rdmas[(0, c)].start()

        for s in range(N_STEPS):
            for c in range(3):
                r = rdmas.pop((s, c))
                r.wait()
                off, ln = CHUNKS[c]
                acc[pl.ds(off, ln)] += recv[s, pl.ds(off, ln)]
                if s + 1 < N_STEPS:
                    rdmas[(s + 1, c)] = make_rdma(s + 1, c)
                    rdmas[(s + 1, c)].start()

        out_ref[...] = acc[...].astype(jnp.float32)

    return pl.pallas_call(
        body,
        out_shape=jax.ShapeDtypeStruct((m, n), jnp.float32),
        in_specs=[pl.BlockSpec(memory_space=pltpu.VMEM)],
        out_specs=pl.BlockSpec(memory_space=pltpu.VMEM),
        scratch_shapes=[
            pltpu.VMEM((m, n), jnp.bfloat16),
            pltpu.VMEM((N_STEPS, m, n), jnp.bfloat16),
            pltpu.SemaphoreType.DMA((N_STEPS, 3)),
            pltpu.SemaphoreType.DMA((N_STEPS, 3)),
        ],
        compiler_params=pltpu.CompilerParams(collective_id=0),
    )(x2)


# device time: 15259 ns/iter; 1.8466x vs baseline; 1.1120x over previous
import jax
import jax.numpy as jnp
from jax import lax
from jax.experimental import pallas as pl
from jax.experimental.pallas import tpu as pltpu

FLOWS = (
    (0, 96, 0), (96, 80, 0),
    (176, 96, 1), (272, 80, 1),
    (352, 80, 2), (432, 80, 2),
)
N_STEPS = 3


def kernel(x):
    m, n = x.shape[-2], x.shape[-1]
    x2 = x.reshape(m, n)

    def body(x_ref, out_ref, acc, recv, send_sems, recv_sems):
        me = (lax.axis_index("x"), lax.axis_index("y"), lax.axis_index("z"))

        def peer_along(a):
            p = list(me)
            p[a] = 1 - p[a]
            return tuple(p)

        barrier_sem = pltpu.get_barrier_semaphore()
        for a in range(3):
            pl.semaphore_signal(
                barrier_sem, inc=1,
                device_id=peer_along(a), device_id_type=pl.DeviceIdType.MESH,
            )
        pl.semaphore_wait(barrier_sem, 3)

        def make_rdma(s, f):
            off, ln, c = FLOWS[f]
            return pltpu.make_async_remote_copy(
                src_ref=acc.at[pl.ds(off, ln)],
                dst_ref=recv.at[s, pl.ds(off, ln)],
                send_sem=send_sems.at[s, f],
                recv_sem=recv_sems.at[s, f],
                device_id=peer_along((c + s) % 3),
                device_id_type=pl.DeviceIdType.MESH,
            )

        rdmas = {}
        for f, (off, ln, _) in enumerate(FLOWS):
            acc[pl.ds(off, ln)] = x_ref[pl.ds(off, ln)].astype(jnp.bfloat16)
            rdmas[(0, f)] = make_rdma(0, f)
            rdmas[(0, f)].start()

        flow_order = (0, 2, 4, 1, 3, 5)
        for s in range(N_STEPS):
            for f in flow_order:
                off, ln, _ = FLOWS[f]
                r = rdmas.pop((s, f))
                r.wait()
                if s + 1 < N_STEPS:
                    acc[pl.ds(off, ln)] += recv[s, pl.ds(off, ln)]
                    rdmas[(s + 1, f)] = make_rdma(s + 1, f)
                    rdmas[(s + 1, f)].start()
                else:
                    out_ref[pl.ds(off, ln)] = (
                        acc[pl.ds(off, ln)].astype(jnp.float32)
                        + recv[s, pl.ds(off, ln)].astype(jnp.float32)
                    )

    return pl.pallas_call(
        body,
        out_shape=jax.ShapeDtypeStruct((m, n), jnp.float32),
        in_specs=[pl.BlockSpec(memory_space=pltpu.VMEM)],
        out_specs=pl.BlockSpec(memory_space=pltpu.VMEM),
        scratch_shapes=[
            pltpu.VMEM((m, n), jnp.bfloat16),
            pltpu.VMEM((N_STEPS, m, n), jnp.bfloat16),
            pltpu.SemaphoreType.DMA((N_STEPS, len(FLOWS))),
            pltpu.SemaphoreType.DMA((N_STEPS, len(FLOWS))),
        ],
        compiler_params=pltpu.CompilerParams(collective_id=0),
    )(x2)


# device time: 14723 ns/iter; 1.9138x vs baseline; 1.0364x over previous
from collections import defaultdict

import jax
import jax.numpy as jnp
from jax import lax
from jax.experimental import pallas as pl
from jax.experimental.pallas import tpu as pltpu

FLOWS = (
    (0, 64, 0), (64, 64, 0), (128, 48, 0),
    (176, 64, 1), (240, 64, 1), (304, 48, 1),
    (352, 64, 2), (416, 48, 2), (464, 48, 2),
)
N_STEPS = 3

_BY_CHUNK = defaultdict(list)
for _f, (_, _, _c) in enumerate(FLOWS):
    _BY_CHUNK[_c].append(_f)
ORDER = tuple(
    _BY_CHUNK[c][h]
    for h in range(max(len(v) for v in _BY_CHUNK.values()))
    for c in sorted(_BY_CHUNK)
    if h < len(_BY_CHUNK[c])
)


def kernel(x):
    m, n = x.shape[-2], x.shape[-1]
    x2 = x.reshape(m, n)

    def body(x_ref, out_ref, acc, recv, send_sems, recv_sems):
        me = (lax.axis_index("x"), lax.axis_index("y"), lax.axis_index("z"))

        def peer_along(a):
            p = list(me)
            p[a] = 1 - p[a]
            return tuple(p)

        barrier_sem = pltpu.get_barrier_semaphore()
        for a in range(3):
            pl.semaphore_signal(
                barrier_sem, inc=1,
                device_id=peer_along(a), device_id_type=pl.DeviceIdType.MESH,
            )
        for off, ln, _ in FLOWS:
            acc[pl.ds(off, ln)] = x_ref[pl.ds(off, ln)].astype(jnp.bfloat16)
        pl.semaphore_wait(barrier_sem, 3)

        def make_rdma(s, f):
            off, ln, c = FLOWS[f]
            return pltpu.make_async_remote_copy(
                src_ref=acc.at[pl.ds(off, ln)],
                dst_ref=recv.at[s, pl.ds(off, ln)],
                send_sem=send_sems.at[s, f],
                recv_sem=recv_sems.at[s, f],
                device_id=peer_along((c + s) % 3),
                device_id_type=pl.DeviceIdType.MESH,
            )

        rdmas = {}
        for f in ORDER:
            rdmas[(0, f)] = make_rdma(0, f)
            rdmas[(0, f)].start()

        last = []
        for s in range(N_STEPS):
            for f in ORDER:
                off, ln, _ = FLOWS[f]
                r = rdmas.pop((s, f))
                if s + 1 < N_STEPS:
                    r.wait()
                    acc[pl.ds(off, ln)] += recv[s, pl.ds(off, ln)]
                    rdmas[(s + 1, f)] = make_rdma(s + 1, f)
                    rdmas[(s + 1, f)].start()
                else:
                    r.wait_recv()
                    last.append(r)
                    out_ref[pl.ds(off, ln)] = (
                        acc[pl.ds(off, ln)].astype(jnp.float32)
                        + recv[s, pl.ds(off, ln)].astype(jnp.float32)
                    )
        for r in last:
            r.wait_send()

    return pl.pallas_call(
        body,
        out_shape=jax.ShapeDtypeStruct((m, n), jnp.float32),
        in_specs=[pl.BlockSpec(memory_space=pltpu.VMEM)],
        out_specs=pl.BlockSpec(memory_space=pltpu.VMEM),
        scratch_shapes=[
            pltpu.VMEM((m, n), jnp.bfloat16),
            pltpu.VMEM((N_STEPS, m, n), jnp.bfloat16),
            pltpu.SemaphoreType.DMA((N_STEPS, len(FLOWS))),
            pltpu.SemaphoreType.DMA((N_STEPS, len(FLOWS))),
        ],
        compiler_params=pltpu.CompilerParams(collective_id=0),
    )(x2)
